# initial kernel scaffold (unmeasured)
import jax
import jax.numpy as jnp
from jax import lax
from jax.experimental import pallas as pl
from jax.experimental.pallas import tpu as pltpu

N_B = 16
N_H = 16
D = 64
HD = N_H * D
SCALE = D ** -0.5


def _body(qm_ref, k_ref, v_ref, out_ref,
          o_acc, ml_acc, o_rx, ml_rx, send_sems, recv_sems):
    b = pl.program_id(0)
    nb = pl.num_programs(0)

    qm = qm_ref[0]
    k2 = k_ref[0].astype(jnp.bfloat16)
    s = lax.dot_general(k2, qm, (((1,), (0,)), ((), ())),
                        preferred_element_type=jnp.float32)
    m = jnp.max(s, axis=0, keepdims=True)
    p = jnp.exp(s - m)
    l = jnp.sum(p, axis=0, keepdims=True)
    pt = p.T.astype(jnp.bfloat16)
    v2 = v_ref[0].astype(jnp.bfloat16)
    obig = lax.dot_general(pt, v2, (((1,), (0,)), ((), ())),
                           preferred_element_type=jnp.float32)
    hh = lax.broadcasted_iota(jnp.int32, (N_H, HD), 0)
    jj = lax.broadcasted_iota(jnp.int32, (N_H, HD), 1)
    opart = jnp.sum(jnp.where(jj // D == hh, obig, 0.0),
                    axis=0, keepdims=True)

    o_acc[pl.ds(b, 1), :] = opart
    ml_acc[0, pl.ds(b, 1), :] = m
    ml_acc[1, pl.ds(b, 1), :] = l

    @pl.when(b == nb - 1)
    def _():
        my_x = lax.axis_index("x")
        my_y = lax.axis_index("y")
        peer = (1 - my_x, my_y)
        rdma_o = pltpu.make_async_remote_copy(
            src_ref=o_acc, dst_ref=o_rx,
            send_sem=send_sems.at[0], recv_sem=recv_sems.at[0],
            device_id=peer, device_id_type=pl.DeviceIdType.MESH)
        rdma_ml = pltpu.make_async_remote_copy(
            src_ref=ml_acc, dst_ref=ml_rx,
            send_sem=send_sems.at[1], recv_sem=recv_sems.at[1],
            device_id=peer, device_id_type=pl.DeviceIdType.MESH)
        rdma_o.start()
        rdma_ml.start()
        rdma_o.wait()
        rdma_ml.wait()

        m_a = ml_acc[0]
        l_a = ml_acc[1]
        m_b = ml_rx[0]
        l_b = ml_rx[1]
        mx = jnp.maximum(m_a, m_b)
        ca = jnp.exp(m_a - mx)
        cb = jnp.exp(m_b - mx)
        den = l_a * ca + l_b * cb
        ca_e = jnp.broadcast_to(ca[:, :, None], (N_B, N_H, D)).reshape(N_B, HD)
        cb_e = jnp.broadcast_to(cb[:, :, None], (N_B, N_H, D)).reshape(N_B, HD)
        den_e = jnp.broadcast_to(den[:, :, None], (N_B, N_H, D)).reshape(N_B, HD)
        num = o_acc[:, :] * ca_e + o_rx[:, :] * cb_e
        out_ref[:, :] = num / den_e


def kernel(Q, K, V):
    b, sq, h, d = Q.shape
    skv = K.shape[1]
    hd = h * d

    q2 = Q.reshape(b, hd)
    col = jnp.arange(hd) // d
    mask = col[:, None] == jnp.arange(h)[None, :]
    qmat = jnp.where(mask[None], q2[:, :, None] * SCALE, 0.0)
    qmat = qmat.astype(jnp.bfloat16)

    k2 = K.reshape(b, skv, hd)
    v2 = V.reshape(b, skv, hd)

    out = pl.pallas_call(
        _body,
        grid=(b,),
        in_specs=[
            pl.BlockSpec((1, hd, h), lambda i: (i, 0, 0)),
            pl.BlockSpec((1, skv, hd), lambda i: (i, 0, 0)),
            pl.BlockSpec((1, skv, hd), lambda i: (i, 0, 0)),
        ],
        out_specs=pl.BlockSpec((b, hd), lambda i: (0, 0)),
        out_shape=jax.ShapeDtypeStruct((b, hd), jnp.float32),
        scratch_shapes=[
            pltpu.VMEM((b, hd), jnp.float32),
            pltpu.VMEM((2, b, h), jnp.float32),
            pltpu.VMEM((b, hd), jnp.float32),
            pltpu.VMEM((2, b, h), jnp.float32),
            pltpu.SemaphoreType.DMA((2,)),
            pltpu.SemaphoreType.DMA((2,)),
        ],
        compiler_params=pltpu.CompilerParams(
            dimension_semantics=("arbitrary",),
            collective_id=0,
        ),
    )(qmat, k2, v2)
    return out.reshape(b, 1, h, d)


# baseline (device time: 185325 ns/iter reference)
import jax
import jax.numpy as jnp
from jax import lax
from jax.experimental import pallas as pl
from jax.experimental.pallas import tpu as pltpu

N_B = 16
N_H = 16
D = 64
HD = N_H * D
SCALE = D ** -0.5


def _body(qm_ref, k_ref, v_ref, out_ref,
          o_acc, ml_acc, o_rx, ml_rx, send_sems, recv_sems):
    b = pl.program_id(0)
    nb = pl.num_programs(0)

    qm = qm_ref[0]
    k2 = k_ref[0].astype(jnp.bfloat16)
    s = lax.dot_general(k2, qm, (((1,), (0,)), ((), ())),
                        preferred_element_type=jnp.float32)
    m = jnp.max(s, axis=0, keepdims=True)
    p = jnp.exp(s - m)
    l = jnp.sum(p, axis=0, keepdims=True)
    pt = p.T.astype(jnp.bfloat16)
    v2 = v_ref[0].astype(jnp.bfloat16)
    obig = lax.dot_general(pt, v2, (((1,), (0,)), ((), ())),
                           preferred_element_type=jnp.float32)
    hh = lax.broadcasted_iota(jnp.int32, (N_H, HD), 0)
    jj = lax.broadcasted_iota(jnp.int32, (N_H, HD), 1)
    opart = jnp.sum(jnp.where(jj // D == hh, obig, 0.0),
                    axis=0, keepdims=True)

    o_acc[pl.ds(b, 1), :] = opart
    ml_acc[0, pl.ds(b, 1), :] = m
    ml_acc[1, pl.ds(b, 1), :] = l

    @pl.when(b == nb - 1)
    def _():
        my_x = lax.axis_index("x")
        my_y = lax.axis_index("y")
        peer = (1 - my_x, my_y)
        rdma_o = pltpu.make_async_remote_copy(
            src_ref=o_acc, dst_ref=o_rx,
            send_sem=send_sems.at[0], recv_sem=recv_sems.at[0],
            device_id=peer, device_id_type=pl.DeviceIdType.MESH)
        rdma_ml = pltpu.make_async_remote_copy(
            src_ref=ml_acc, dst_ref=ml_rx,
            send_sem=send_sems.at[1], recv_sem=recv_sems.at[1],
            device_id=peer, device_id_type=pl.DeviceIdType.MESH)
        rdma_o.start()
        rdma_ml.start()
        rdma_o.wait()
        rdma_ml.wait()

        m_a = ml_acc[0]
        l_a = ml_acc[1]
        m_b = ml_rx[0]
        l_b = ml_rx[1]
        mx = jnp.maximum(m_a, m_b)
        ca = jnp.exp(m_a - mx)
        cb = jnp.exp(m_b - mx)
        den = l_a * ca + l_b * cb
        ca_e = jnp.broadcast_to(ca[:, :, None], (N_B, N_H, D)).reshape(N_B, HD)
        cb_e = jnp.broadcast_to(cb[:, :, None], (N_B, N_H, D)).reshape(N_B, HD)
        den_e = jnp.broadcast_to(den[:, :, None], (N_B, N_H, D)).reshape(N_B, HD)
        num = o_acc[:, :] * ca_e + o_rx[:, :] * cb_e
        out_ref[:, :] = num / den_e


def kernel(Q, K, V):
    b, sq, h, d = Q.shape
    skv = K.shape[1]
    hd = h * d

    q2 = Q.reshape(b, hd)
    col = jnp.arange(hd) // d
    mask = col[:, None] == jnp.arange(h)[None, :]
    qmat = jnp.where(mask[None], q2[:, :, None] * SCALE, 0.0)
    qmat = qmat.astype(jnp.bfloat16)

    k2 = K.reshape(b, skv, hd)
    v2 = V.reshape(b, skv, hd)

    out = pl.pallas_call(
        _body,
        grid=(b,),
        in_specs=[
            pl.BlockSpec((1, hd, h), lambda i: (i, 0, 0)),
            pl.BlockSpec((1, skv, hd), lambda i: (i, 0, 0)),
            pl.BlockSpec((1, skv, hd), lambda i: (i, 0, 0)),
        ],
        out_specs=pl.BlockSpec((b, hd), lambda i: (0, 0)),
        out_shape=jax.ShapeDtypeStruct((b, hd), jnp.float32),
        scratch_shapes=[
            pltpu.VMEM((b, hd), jnp.float32),
            pltpu.VMEM((2, b, h), jnp.float32),
            pltpu.VMEM((b, hd), jnp.float32),
            pltpu.VMEM((2, b, h), jnp.float32),
            pltpu.SemaphoreType.DMA((2,)),
            pltpu.SemaphoreType.DMA((2,)),
        ],
        compiler_params=pltpu.CompilerParams(
            dimension_semantics=("arbitrary",),
        ),
    )(qmat, k2, v2)
    return out.reshape(b, 1, h, d)


# device time: 165348 ns/iter; 1.1208x vs baseline; 1.1208x over previous
import jax
import jax.numpy as jnp
from jax import lax
from jax.experimental import pallas as pl
from jax.experimental.pallas import tpu as pltpu

N_B = 16
N_H = 16
D = 64
HD = N_H * D
B_LOC = 8
SCALE = D ** -0.5


def _body(boff_ref, qm_ref, k_ref, v_ref, out_ref,
          o_acc, ml_acc, o_rx, ml_rx, send_sems, recv_sems):
    i = pl.program_id(0)
    ni = pl.num_programs(0)

    qm = qm_ref[0]
    k2 = k_ref[0].astype(jnp.bfloat16)
    s = lax.dot_general(k2, qm, (((1,), (0,)), ((), ())),
                        preferred_element_type=jnp.float32)
    m = jnp.max(s, axis=0, keepdims=True)
    p = jnp.exp(s - m)
    l = jnp.sum(p, axis=0, keepdims=True)
    pt = p.T.astype(jnp.bfloat16)
    v2 = v_ref[0].astype(jnp.bfloat16)
    obig = lax.dot_general(pt, v2, (((1,), (0,)), ((), ())),
                           preferred_element_type=jnp.float32)
    hh = lax.broadcasted_iota(jnp.int32, (N_H, HD), 0)
    jj = lax.broadcasted_iota(jnp.int32, (N_H, HD), 1)
    opart = jnp.sum(jnp.where(jj // D == hh, obig, 0.0),
                    axis=0, keepdims=True)

    o_acc[pl.ds(i, 1), :] = opart
    ml_acc[0, pl.ds(i, 1), :] = m
    ml_acc[1, pl.ds(i, 1), :] = l

    @pl.when(i == ni - 1)
    def _():
        my_x = lax.axis_index("x")
        my_y = lax.axis_index("y")
        boff = pl.multiple_of(boff_ref[0], B_LOC)

        peer_x = (1 - my_x, my_y)
        rdma_o = pltpu.make_async_remote_copy(
            src_ref=o_acc, dst_ref=o_rx,
            send_sem=send_sems.at[0], recv_sem=recv_sems.at[0],
            device_id=peer_x, device_id_type=pl.DeviceIdType.MESH)
        rdma_ml = pltpu.make_async_remote_copy(
            src_ref=ml_acc, dst_ref=ml_rx,
            send_sem=send_sems.at[1], recv_sem=recv_sems.at[1],
            device_id=peer_x, device_id_type=pl.DeviceIdType.MESH)
        rdma_o.start()
        rdma_ml.start()
        rdma_o.wait()
        rdma_ml.wait()

        m_a = ml_acc[0]
        l_a = ml_acc[1]
        m_b = ml_rx[0]
        l_b = ml_rx[1]
        mx = jnp.maximum(m_a, m_b)
        ca = jnp.exp(m_a - mx)
        cb = jnp.exp(m_b - mx)
        den = l_a * ca + l_b * cb
        ca_e = jnp.broadcast_to(
            ca[:, :, None], (B_LOC, N_H, D)).reshape(B_LOC, HD)
        cb_e = jnp.broadcast_to(
            cb[:, :, None], (B_LOC, N_H, D)).reshape(B_LOC, HD)
        den_e = jnp.broadcast_to(
            den[:, :, None], (B_LOC, N_H, D)).reshape(B_LOC, HD)
        num = o_acc[:, :] * ca_e + o_rx[:, :] * cb_e
        out_ref[pl.ds(boff, B_LOC), :] = num / den_e

        peer_y = (my_x, 1 - my_y)
        rdma_y = pltpu.make_async_remote_copy(
            src_ref=out_ref.at[pl.ds(boff, B_LOC), :],
            dst_ref=out_ref.at[pl.ds(boff, B_LOC), :],
            send_sem=send_sems.at[2], recv_sem=recv_sems.at[2],
            device_id=peer_y, device_id_type=pl.DeviceIdType.MESH)
        rdma_y.start()
        rdma_y.wait()


def kernel(Q, K, V):
    b, sq, h, d = Q.shape
    skv = K.shape[1]
    hd = h * d
    b_loc = b // 2

    my_y = lax.axis_index("y")
    boff = my_y * b_loc

    q2 = lax.dynamic_slice_in_dim(Q.reshape(b, hd), boff, b_loc, axis=0)
    col = jnp.arange(hd) // d
    mask = col[:, None] == jnp.arange(h)[None, :]
    qmat = jnp.where(mask[None], q2[:, :, None] * SCALE, 0.0)
    qmat = qmat.astype(jnp.bfloat16)

    k2 = K.reshape(b, skv, hd)
    v2 = V.reshape(b, skv, hd)

    grid_spec = pltpu.PrefetchScalarGridSpec(
        num_scalar_prefetch=1,
        grid=(b_loc,),
        in_specs=[
            pl.BlockSpec((1, hd, h), lambda i, boff: (i, 0, 0)),
            pl.BlockSpec((1, skv, hd), lambda i, boff: (i + boff[0], 0, 0)),
            pl.BlockSpec((1, skv, hd), lambda i, boff: (i + boff[0], 0, 0)),
        ],
        out_specs=pl.BlockSpec((b, hd), lambda i, boff: (0, 0)),
        scratch_shapes=[
            pltpu.VMEM((b_loc, hd), jnp.float32),
            pltpu.VMEM((2, b_loc, h), jnp.float32),
            pltpu.VMEM((b_loc, hd), jnp.float32),
            pltpu.VMEM((2, b_loc, h), jnp.float32),
            pltpu.SemaphoreType.DMA((3,)),
            pltpu.SemaphoreType.DMA((3,)),
        ],
    )
    out = pl.pallas_call(
        _body,
        grid_spec=grid_spec,
        out_shape=jax.ShapeDtypeStruct((b, hd), jnp.float32),
        compiler_params=pltpu.CompilerParams(
            dimension_semantics=("arbitrary",),
        ),
    )(jnp.full((1,), boff, jnp.int32), qmat, k2, v2)
    return out.reshape(b, 1, h, d)


# device time: 34391 ns/iter; 5.3888x vs baseline; 4.8079x over previous
import jax
import jax.numpy as jnp
from jax import lax
from jax.experimental import pallas as pl
from jax.experimental.pallas import tpu as pltpu

N_H = 16
D = 64
SCALE = D ** -0.5


def _body(boff_ref, q_ref, k_ref, v_ref, out_ref,
          o_acc, ml_acc, o_rx, ml_rx, send_sems, recv_sems):
    i = pl.program_id(0)
    ni = pl.num_programs(0)
    b_loc = ni

    q = q_ref[0]
    kt = k_ref[0]
    s = jnp.sum(kt * q[:, :, None], axis=1)
    m = jnp.max(s, axis=1, keepdims=True)
    p = jnp.exp(s - m)
    l = jnp.sum(p, axis=1, keepdims=True)
    vt = v_ref[0]
    o = jnp.sum(vt * p[:, None, :], axis=2)

    o_acc[pl.ds(i, 1)] = o[None]
    ml_acc[0, pl.ds(i, 1), :] = m.reshape(1, N_H)
    ml_acc[1, pl.ds(i, 1), :] = l.reshape(1, N_H)

    @pl.when(i == ni - 1)
    def _():
        my_x = lax.axis_index("x")
        my_y = lax.axis_index("y")
        boff = pl.multiple_of(boff_ref[0], b_loc)

        peer_x = (1 - my_x, my_y)
        rdma_o = pltpu.make_async_remote_copy(
            src_ref=o_acc, dst_ref=o_rx,
            send_sem=send_sems.at[0], recv_sem=recv_sems.at[0],
            device_id=peer_x, device_id_type=pl.DeviceIdType.MESH)
        rdma_ml = pltpu.make_async_remote_copy(
            src_ref=ml_acc, dst_ref=ml_rx,
            send_sem=send_sems.at[1], recv_sem=recv_sems.at[1],
            device_id=peer_x, device_id_type=pl.DeviceIdType.MESH)
        rdma_o.start()
        rdma_ml.start()
        rdma_o.wait()
        rdma_ml.wait()

        m_a = ml_acc[0]
        l_a = ml_acc[1]
        m_b = ml_rx[0]
        l_b = ml_rx[1]
        mx = jnp.maximum(m_a, m_b)
        ca = jnp.exp(m_a - mx)
        cb = jnp.exp(m_b - mx)
        den = l_a * ca + l_b * cb
        num = o_acc[...] * ca[:, :, None] + o_rx[...] * cb[:, :, None]
        out_ref[pl.ds(boff, b_loc)] = num / den[:, :, None]

        peer_y = (my_x, 1 - my_y)
        rdma_y = pltpu.make_async_remote_copy(
            src_ref=out_ref.at[pl.ds(boff, b_loc)],
            dst_ref=out_ref.at[pl.ds(boff, b_loc)],
            send_sem=send_sems.at[2], recv_sem=recv_sems.at[2],
            device_id=peer_y, device_id_type=pl.DeviceIdType.MESH)
        rdma_y.start()
        rdma_y.wait()


def kernel(Q, K, V):
    b, sq, h, d = Q.shape
    skv = K.shape[1]
    b_loc = b // 2

    my_y = lax.axis_index("y")
    boff = my_y * b_loc

    kt = jnp.transpose(K, (0, 2, 3, 1))
    vt = jnp.transpose(V, (0, 2, 3, 1))
    q3 = Q.reshape(b, h, d) * SCALE
    qs = lax.dynamic_slice_in_dim(q3, boff, b_loc, axis=0)

    grid_spec = pltpu.PrefetchScalarGridSpec(
        num_scalar_prefetch=1,
        grid=(b_loc,),
        in_specs=[
            pl.BlockSpec((1, h, d), lambda i, boff: (i, 0, 0)),
            pl.BlockSpec((1, h, d, skv),
                         lambda i, boff: (i + boff[0], 0, 0, 0)),
            pl.BlockSpec((1, h, d, skv),
                         lambda i, boff: (i + boff[0], 0, 0, 0)),
        ],
        out_specs=pl.BlockSpec((b, h, d), lambda i, boff: (0, 0, 0)),
        scratch_shapes=[
            pltpu.VMEM((b_loc, h, d), jnp.float32),
            pltpu.VMEM((2, b_loc, h), jnp.float32),
            pltpu.VMEM((b_loc, h, d), jnp.float32),
            pltpu.VMEM((2, b_loc, h), jnp.float32),
            pltpu.SemaphoreType.DMA((3,)),
            pltpu.SemaphoreType.DMA((3,)),
        ],
    )
    out = pl.pallas_call(
        _body,
        grid_spec=grid_spec,
        out_shape=jax.ShapeDtypeStruct((b, h, d), jnp.float32),
        compiler_params=pltpu.CompilerParams(
            dimension_semantics=("arbitrary",),
        ),
    )(jnp.full((1,), boff, jnp.int32), qs, kt, vt)
    return out.reshape(b, 1, h, d)
